# Initial kernel scaffold; baseline (speedup 1.0000x reference)
import jax
import jax.numpy as jnp
from jax.experimental import pallas as pl


def kernel(x, table):
    raise NotImplementedError("write your pallas kernel here")


# speedup vs baseline: 1.7969x; 1.7969x over previous
import jax
import jax.numpy as jnp
from jax import lax
from jax.experimental import pallas as pl
from jax.experimental.pallas import tpu as pltpu
from jax.experimental.pallas import tpu_sc as plsc

_DIM = 64
_NC = 2
_NS = 16
_NW = _NC * _NS
_CHUNK = 512


def _emb_body(x_hbm, table_hbm, out_hbm, idx_v, rows_v, sem):
    wid = lax.axis_index("s") * _NC + lax.axis_index("c")
    b_per_w = x_hbm.shape[0] // _NW
    base = wid * b_per_w
    nsteps = b_per_w // _CHUNK

    @pl.loop(0, nsteps)
    def _step(i):
        off = base + i * _CHUNK
        pltpu.sync_copy(x_hbm.at[pl.ds(off, _CHUNK)], idx_v)
        pltpu.async_copy(table_hbm.at[idx_v], rows_v, sem).wait()
        pltpu.sync_copy(rows_v, out_hbm.at[pl.ds(off, _CHUNK)])


def kernel(x, table):
    B, L = x.shape
    n = B * L
    xf = x.reshape(n).astype(jnp.int32)
    mesh = plsc.VectorSubcoreMesh(core_axis_name="c", subcore_axis_name="s")
    k = pl.kernel(
        _emb_body,
        out_type=jax.ShapeDtypeStruct((n, _DIM), jnp.float32),
        mesh=mesh,
        scratch_types=[
            pltpu.VMEM((_CHUNK,), jnp.int32),
            pltpu.VMEM((_CHUNK, _DIM), jnp.float32),
            pltpu.SemaphoreType.DMA,
        ],
        compiler_params=pltpu.CompilerParams(use_tc_tiling_on_sc=False),
    )
    out = k(xf, table)
    return out.reshape(B, L, _DIM)


# speedup vs baseline: 1.8745x; 1.0432x over previous
import jax
import jax.numpy as jnp
from jax import lax
from jax.experimental import pallas as pl
from jax.experimental.pallas import tpu as pltpu
from jax.experimental.pallas import tpu_sc as plsc

_DIM = 64
_NC = 2
_NS = 16
_NW = _NC * _NS
_CHUNK = 400
_NBUF = 4


def _emb_body(x_hbm, table_hbm, out_hbm, idx_all, rows, sems):
    wid = lax.axis_index("s") * _NC + lax.axis_index("c")
    b_per_w = x_hbm.shape[0] // _NW
    base = wid * b_per_w
    nsteps = b_per_w // _CHUNK
    ngroups = nsteps // _NBUF

    pltpu.sync_copy(x_hbm.at[pl.ds(base, b_per_w)], idx_all)

    def start_gather(step, b):
        pltpu.async_copy(
            table_hbm.at[idx_all.at[pl.ds(step * _CHUNK, _CHUNK)]],
            rows[b], sems[b])

    def wait_gather(step, b):
        pltpu.make_async_copy(
            table_hbm.at[idx_all.at[pl.ds(step * _CHUNK, _CHUNK)]],
            rows[b], sems[b]).wait()

    def write_out(step, b):
        pltpu.sync_copy(rows[b], out_hbm.at[pl.ds(base + step * _CHUNK, _CHUNK)])

    for b in range(_NBUF):
        start_gather(b, b)

    @pl.loop(0, ngroups - 1)
    def _group(g):
        for b in range(_NBUF):
            i = g * _NBUF + b
            wait_gather(i, b)
            write_out(i, b)
            start_gather(i + _NBUF, b)

    for b in range(_NBUF):
        i = (ngroups - 1) * _NBUF + b
        wait_gather(i, b)
        write_out(i, b)


def kernel(x, table):
    B, L = x.shape
    n = B * L
    xf = x.reshape(n).astype(jnp.int32)
    b_per_w = n // _NW
    mesh = plsc.VectorSubcoreMesh(core_axis_name="c", subcore_axis_name="s")
    k = pl.kernel(
        _emb_body,
        out_type=jax.ShapeDtypeStruct((n, _DIM), jnp.float32),
        mesh=mesh,
        scratch_types=[
            pltpu.VMEM((b_per_w,), jnp.int32),
            [pltpu.VMEM((_CHUNK, _DIM), jnp.float32) for _ in range(_NBUF)],
            [pltpu.SemaphoreType.DMA for _ in range(_NBUF)],
        ],
        compiler_params=pltpu.CompilerParams(use_tc_tiling_on_sc=False),
    )
    out = k(xf, table)
    return out.reshape(B, L, _DIM)
